# baseline (speedup 1.0000x reference)
import functools

import jax
import jax.numpy as jnp
from jax import lax
from jax.experimental import pallas as pl
from jax.experimental.pallas import tpu as pltpu
from jax.experimental.pallas import tpu_sc as plsc

N = 10000
E = 320000
D = 128
HD = D // 2
CHUNK = 128
NCHUNKS = 2560
EPAD = NCHUNKS * CHUNK
NPAD = 10240
DUMMY = N
ACC_N = 10240
NSUB = 16
NCORE = 2
ROWS = NCHUNKS // NSUB
NBLK = 4
RB = ROWS // NBLK
WB_STRIDE = 624
WB_WIN = 640
MBLK = 1000


def _leaky(a):
    return jnp.maximum(a, 0.2 * a)


def _mm_body(nf_ref, w_ref, wqk_ref, x_ref, qk_ref):
    x = jnp.dot(nf_ref[...], w_ref[...], preferred_element_type=jnp.float32)
    x_ref[0] = x[:, :HD]
    x_ref[1] = x[:, HD:]
    qk_ref[...] = jnp.dot(x, wqk_ref[...], preferred_element_type=jnp.float32)


def _ep_body(p_ref, b_ref, o_ref):
    o_ref[...] = (
        jnp.concatenate([p_ref[0], p_ref[1]], axis=-1) + b_ref[...])


_sc_mesh = plsc.VectorSubcoreMesh(core_axis_name="c", subcore_axis_name="s")


@functools.partial(
    pl.kernel,
    mesh=_sc_mesh,
    compiler_params=pltpu.CompilerParams(
        needs_layout_passes=False, use_tc_tiling_on_sc=False),
    out_type=jax.ShapeDtypeStruct((NCORE, N, HD), jnp.float32),
    scratch_types=[
        pltpu.VMEM((NPAD,), jnp.float32),
        pltpu.VMEM((NPAD,), jnp.float32),
        pltpu.VMEM((RB, CHUNK), jnp.int32),
        pltpu.VMEM((RB, CHUNK), jnp.int32),
        pltpu.VMEM((ROWS, CHUNK), jnp.float32),
        pltpu.VMEM((CHUNK,), jnp.float32),
        pltpu.VMEM((CHUNK, HD), jnp.float32),
        pltpu.VMEM((CHUNK, HD), jnp.float32),
        pltpu.VMEM((CHUNK, HD), jnp.float32),
        pltpu.VMEM((CHUNK, HD), jnp.float32),
        pltpu.VMEM((NPAD // NSUB,), jnp.float32),
        pltpu.VMEM_SHARED((ACC_N, HD), jnp.float32),
        pltpu.VMEM_SHARED((NPAD,), jnp.float32),
        pltpu.SemaphoreType.DMA,
        pltpu.SemaphoreType.DMA,
        pltpu.SemaphoreType.DMA,
        pltpu.SemaphoreType.DMA,
        pltpu.SemaphoreType.DMA,
        pltpu.SemaphoreType.DMA,
    ],
)
def _sc_gat(x_hbm, q_hbm, k_hbm, row_hbm, col_hbm, out_hbm,
            q_v, k_v, row_v, col_v, w2_v, w_v, xr0, xr1, xr2, xr3, z_v,
            acc_sh, den_sh, sem_g, sem_s, s0, s1, s2, s3):
    cid = lax.axis_index("c")
    sid = lax.axis_index("s")

    pltpu.sync_copy(q_hbm, q_v)
    pltpu.sync_copy(k_hbm, k_v)

    def _mx(i, mv):
        return jnp.maximum(mv, q_v[pl.ds(i * 16, 16)])

    def _mxk(i, mv):
        return jnp.maximum(mv, k_v[pl.ds(i * 16, 16)])

    _dn = lax.GatherDimensionNumbers(
        offset_dims=(), collapsed_slice_dims=(0,), start_index_map=(0,))

    def _shuffle(mv, idx):
        return lax.gather(mv, idx[:, None], _dn, slice_sizes=(1,),
                          mode=lax.GatherScatterMode.PROMISE_IN_BOUNDS)

    def _lanemax(mv):
        lanes = lax.iota(jnp.int32, 16)
        for off in (8, 4, 2, 1):
            mv = jnp.maximum(mv, _shuffle(mv, lanes ^ off))
        return mv

    neg = jnp.full((16,), -3.0e38, jnp.float32)
    mq = _lanemax(lax.fori_loop(0, NPAD // 16, _mx, neg))
    mk = _lanemax(lax.fori_loop(0, NPAD // 16, _mxk, neg))
    shift = _leaky(mq + mk)

    zvec = jnp.zeros((16,), jnp.float32)

    def _zrow(i, _):
        for v in range(HD // 16):
            xr0[i, pl.ds(v * 16, 16)] = zvec
        return 0

    lax.fori_loop(0, CHUNK, _zrow, 0)

    def _zv(i, _):
        z_v[pl.ds(i * 16, 16)] = zvec
        return 0

    lax.fori_loop(0, (NPAD // NSUB) // 16, _zv, 0)

    acc_rows_per_tile = ACC_N // NSUB
    for t in range(acc_rows_per_tile // CHUNK):
        pltpu.sync_copy(
            xr0, acc_sh.at[pl.ds(sid * acc_rows_per_tile + t * CHUNK, CHUNK)])
    pltpu.sync_copy(z_v, den_sh.at[pl.ds(sid * (NPAD // NSUB), NPAD // NSUB)])
    plsc.subcore_barrier()

    base = sid * ROWS

    def _p1blk(blk, _):
        bb = base + blk * RB
        pltpu.sync_copy(row_hbm.at[pl.ds(bb, RB)], row_v)
        pltpu.sync_copy(col_hbm.at[pl.ds(bb, RB)], col_v)

        def _p1(c, _2):
            g = blk * RB + c
            for v in range(CHUNK // 16):
                sl = pl.ds(v * 16, 16)
                rv = row_v[c, sl]
                cv = col_v[c, sl]
                qg = plsc.load_gather(q_v, [rv])
                kg = plsc.load_gather(k_v, [cv])
                w2_v[g, sl] = jnp.exp(_leaky(qg + kg) - shift)
            pltpu.async_copy(
                w2_v.at[g], den_sh.at[row_v.at[c]], sem_s, add=True)
            return 0

        lax.fori_loop(0, RB, _p1, 0)

        def _p1drain(c, _2):
            pltpu.make_async_copy(
                w2_v.at[blk * RB + c], den_sh.at[row_v.at[c]], sem_s).wait()
            return 0

        lax.fori_loop(0, RB, _p1drain, 0)
        return 0

    with jax.named_scope("sc_pass1"):
        lax.fori_loop(0, NBLK, _p1blk, 0)
        plsc.subcore_barrier()

    with jax.named_scope("sc_inv"):
        pltpu.sync_copy(den_sh, q_v)

        def _inv(i, _):
            sl = pl.ds(i * 16, 16)
            q_v[sl] = 1.0 / q_v[sl]
            return 0

        lax.fori_loop(0, NPAD // 16, _inv, 0)

    xrs = (xr0, xr1, xr2, xr3)
    sems = (s0, s1, s2, s3)

    def _gather(c, slot):
        pltpu.async_copy(
            x_hbm.at[cid].at[col_v.at[c]], xrs[slot], sems[slot])

    def _wait_slot(c, slot):
        pltpu.make_async_copy(
            x_hbm.at[cid].at[col_v.at[c]], xrs[slot], sems[slot]).wait()

    def _p2chunk(blk, c, slot, prefetch, drain_first):
        if prefetch:
            if drain_first:
                _wait_slot(c, (slot + 2) % 4)
            _gather(c + 2, (slot + 2) % 4)
        _wait_slot(c, slot)
        xr = xrs[slot]
        g = blk * RB + c
        for v in range(CHUNK // 16):
            sl = pl.ds(v * 16, 16)
            rv = row_v[c, sl]
            iv = plsc.load_gather(q_v, [rv])
            w_v[sl] = w2_v[g, sl] * iv

        def _scale(e2, _2):
            for u in range(2):
                e = e2 * 2 + u
                cb = plsc.load_gather(w_v, [lax.broadcast(e, (16,))])
                for v in range(HD // 16):
                    sl = pl.ds(v * 16, 16)
                    xr[e, sl] = xr[e, sl] * cb
            return 0

        lax.fori_loop(0, CHUNK // 2, _scale, 0)
        pltpu.async_copy(xr, acc_sh.at[row_v.at[c]], sems[slot], add=True)

    def _p2blk(blk):
        bb = base + blk * RB
        pltpu.sync_copy(row_hbm.at[pl.ds(bb, RB)], row_v)
        pltpu.sync_copy(col_hbm.at[pl.ds(bb, RB)], col_v)
        _gather(0, 0)
        _gather(1, 1)
        for b in range(4):
            _p2chunk(blk, b, b, prefetch=True, drain_first=(b >= 2))

        def _p2(i, _2):
            for b in range(4):
                _p2chunk(blk, i * 4 + b, b, prefetch=True, drain_first=True)
            return 0

        lax.fori_loop(1, RB // 4 - 1, _p2, 0)
        for b in range(4):
            _p2chunk(blk, RB - 4 + b, b, prefetch=(b < 2), drain_first=(b < 2))
        for b in range(4):
            _wait_slot(RB - 4 + b, b)

    with jax.named_scope("sc_pass2"):
        for blk in range(NBLK):
            _p2blk(blk)
        plsc.subcore_barrier()

    wb = sid * WB_STRIDE
    nwb = WB_WIN // CHUNK

    def _wb_slice(t):
        return pl.ds(wb + t * CHUNK, CHUNK)

    def _wb_buf(t):
        return xr0 if t % 2 == 0 else xr1

    with jax.named_scope("sc_wb"):
        for t in range(nwb):
            if t >= 2:
                pltpu.make_async_copy(
                    _wb_buf(t - 2), out_hbm.at[cid].at[_wb_slice(t - 2)],
                    sem_g).wait()
            pltpu.sync_copy(acc_sh.at[_wb_slice(t)], _wb_buf(t))
            pltpu.async_copy(
                _wb_buf(t), out_hbm.at[cid].at[_wb_slice(t)], sem_g)
        for t in range(nwb - 2, nwb):
            pltpu.make_async_copy(
                _wb_buf(t), out_hbm.at[cid].at[_wb_slice(t)], sem_g).wait()


def kernel(node_features, edge_index, is_training, W_values, w_query, w_key, b):
    f32 = jnp.float32
    wqk = jnp.concatenate(
        [w_query, w_key, jnp.zeros((D, D - 2), f32)], axis=1)

    x2, qk = pl.pallas_call(
        _mm_body,
        grid=(N // MBLK,),
        in_specs=[
            pl.BlockSpec((MBLK, D), lambda i: (i, 0)),
            pl.BlockSpec((D, D), lambda i: (0, 0)),
            pl.BlockSpec((D, D), lambda i: (0, 0)),
        ],
        out_specs=[
            pl.BlockSpec((NCORE, MBLK, HD), lambda i: (0, i, 0)),
            pl.BlockSpec((MBLK, D), lambda i: (i, 0)),
        ],
        out_shape=[
            jax.ShapeDtypeStruct((NCORE, N, HD), f32),
            jax.ShapeDtypeStruct((N, D), f32),
        ],
    )(node_features, W_values, wqk)

    q = jnp.pad(qk[:, 0], (0, NPAD - N))
    k = jnp.pad(qk[:, 1], (0, NPAD - N))

    npad = EPAD - E
    row = jnp.concatenate(
        [edge_index[0], jnp.full((npad,), DUMMY, jnp.int32)]).reshape(
            NCHUNKS, CHUNK)
    col = jnp.concatenate(
        [edge_index[1], jnp.zeros((npad,), jnp.int32)]).reshape(
            NCHUNKS, CHUNK)

    parts = _sc_gat(x2, q, k, row, col)

    out = pl.pallas_call(
        _ep_body,
        grid=(N // MBLK,),
        in_specs=[
            pl.BlockSpec((NCORE, MBLK, HD), lambda i: (0, i, 0)),
            pl.BlockSpec((1, D), lambda i: (0, 0)),
        ],
        out_specs=pl.BlockSpec((MBLK, D), lambda i: (i, 0)),
        out_shape=jax.ShapeDtypeStruct((N, D), f32),
    )(parts, b.reshape(1, D))
    return out


# speedup vs baseline: 1.0731x; 1.0731x over previous
import functools

import jax
import jax.numpy as jnp
from jax import lax
from jax.experimental import pallas as pl
from jax.experimental.pallas import tpu as pltpu
from jax.experimental.pallas import tpu_sc as plsc

N = 10000
E = 320000
D = 128
HD = D // 2
CHUNK = 128
NCHUNKS = 2560
EPAD = NCHUNKS * CHUNK
NPAD = 10240
DUMMY = N
ACC_N = 10240
NSUB = 16
NCORE = 2
ROWS = NCHUNKS // NSUB
NBLK = 4
RB = ROWS // NBLK
WB_STRIDE = 624
WB_WIN = 640
MBLK = 1000


def _leaky(a):
    return jnp.maximum(a, 0.2 * a)


def _mm_body(nf_ref, w_ref, wqk_ref, x_ref, qk_ref):
    x = jnp.dot(nf_ref[...], w_ref[...], preferred_element_type=jnp.float32)
    x_ref[0] = x[:, :HD]
    x_ref[1] = x[:, HD:]
    qk_ref[...] = jnp.dot(x, wqk_ref[...], preferred_element_type=jnp.float32)


def _ep_body(p_ref, b_ref, o_ref):
    o_ref[...] = (
        jnp.concatenate([p_ref[0], p_ref[1]], axis=-1) + b_ref[...])


_sc_mesh = plsc.VectorSubcoreMesh(core_axis_name="c", subcore_axis_name="s")


@functools.partial(
    pl.kernel,
    mesh=_sc_mesh,
    compiler_params=pltpu.CompilerParams(
        needs_layout_passes=False, use_tc_tiling_on_sc=False),
    out_type=jax.ShapeDtypeStruct((NCORE, N, HD), jnp.float32),
    scratch_types=[
        pltpu.VMEM((NPAD,), jnp.float32),
        pltpu.VMEM((NPAD,), jnp.float32),
        pltpu.VMEM((RB, CHUNK), jnp.int32),
        pltpu.VMEM((RB, CHUNK), jnp.int32),
        pltpu.VMEM((ROWS, CHUNK), jnp.float32),
        pltpu.VMEM((CHUNK,), jnp.float32),
        pltpu.VMEM((CHUNK, HD), jnp.float32),
        pltpu.VMEM((CHUNK, HD), jnp.float32),
        pltpu.VMEM((CHUNK, HD), jnp.float32),
        pltpu.VMEM((CHUNK, HD), jnp.float32),
        pltpu.VMEM((NPAD // NSUB,), jnp.float32),
        pltpu.VMEM_SHARED((ACC_N, HD), jnp.float32),
        pltpu.VMEM_SHARED((NPAD,), jnp.float32),
        pltpu.SemaphoreType.DMA,
        pltpu.SemaphoreType.DMA,
        pltpu.SemaphoreType.DMA,
        pltpu.SemaphoreType.DMA,
        pltpu.SemaphoreType.DMA,
        pltpu.SemaphoreType.DMA,
    ],
)
def _sc_gat(x_hbm, q_hbm, k_hbm, row_hbm, col_hbm, out_hbm,
            q_v, k_v, row_v, col_v, w2_v, w_v, xr0, xr1, xr2, xr3, z_v,
            acc_sh, den_sh, sem_g, sem_s, s0, s1, s2, s3):
    cid = lax.axis_index("c")
    sid = lax.axis_index("s")

    with jax.named_scope("sc_setup"):
        pltpu.sync_copy(q_hbm, q_v)
        pltpu.sync_copy(k_hbm, k_v)

    def _mx(i, mv):
        return jnp.maximum(mv, q_v[pl.ds(i * 16, 16)])

    def _mxk(i, mv):
        return jnp.maximum(mv, k_v[pl.ds(i * 16, 16)])

    _dn = lax.GatherDimensionNumbers(
        offset_dims=(), collapsed_slice_dims=(0,), start_index_map=(0,))

    def _shuffle(mv, idx):
        return lax.gather(mv, idx[:, None], _dn, slice_sizes=(1,),
                          mode=lax.GatherScatterMode.PROMISE_IN_BOUNDS)

    def _lanemax(mv):
        lanes = lax.iota(jnp.int32, 16)
        for off in (8, 4, 2, 1):
            mv = jnp.maximum(mv, _shuffle(mv, lanes ^ off))
        return mv

    neg = jnp.full((16,), -3.0e38, jnp.float32)
    mq = _lanemax(lax.fori_loop(0, NPAD // 16, _mx, neg))
    mk = _lanemax(lax.fori_loop(0, NPAD // 16, _mxk, neg))
    shift = _leaky(mq + mk)

    zvec = jnp.zeros((16,), jnp.float32)

    def _zrow(i, _):
        for v in range(HD // 16):
            xr0[i, pl.ds(v * 16, 16)] = zvec
        return 0

    lax.fori_loop(0, CHUNK, _zrow, 0)

    def _zv(i, _):
        z_v[pl.ds(i * 16, 16)] = zvec
        return 0

    lax.fori_loop(0, (NPAD // NSUB) // 16, _zv, 0)

    acc_rows_per_tile = ACC_N // NSUB
    for t in range(acc_rows_per_tile // CHUNK):
        pltpu.sync_copy(
            xr0, acc_sh.at[pl.ds(sid * acc_rows_per_tile + t * CHUNK, CHUNK)])
    pltpu.sync_copy(z_v, den_sh.at[pl.ds(sid * (NPAD // NSUB), NPAD // NSUB)])
    plsc.subcore_barrier()

    base = sid * ROWS

    def _p1blk(blk, _):
        bb = base + blk * RB
        pltpu.sync_copy(row_hbm.at[pl.ds(bb, RB)], row_v)
        pltpu.sync_copy(col_hbm.at[pl.ds(bb, RB)], col_v)

        def _p1(c, _2):
            g = blk * RB + c
            for v in range(CHUNK // 16):
                sl = pl.ds(v * 16, 16)
                rv = row_v[c, sl]
                cv = col_v[c, sl]
                qg = plsc.load_gather(q_v, [rv])
                kg = plsc.load_gather(k_v, [cv])
                w2_v[g, sl] = jnp.exp(_leaky(qg + kg) - shift)
            pltpu.async_copy(
                w2_v.at[g], den_sh.at[row_v.at[c]], sem_s, add=True)
            return 0

        lax.fori_loop(0, RB, _p1, 0)

        def _p1drain(c, _2):
            pltpu.make_async_copy(
                w2_v.at[blk * RB + c], den_sh.at[row_v.at[c]], sem_s).wait()
            return 0

        lax.fori_loop(0, RB, _p1drain, 0)
        return 0

    with jax.named_scope("sc_pass1"):
        lax.fori_loop(0, NBLK, _p1blk, 0)
        plsc.subcore_barrier()

    with jax.named_scope("sc_inv"):
        pltpu.sync_copy(den_sh, q_v)

        @plsc.parallel_loop(0, NPAD // 16, unroll=4)
        def _inv(i):
            sl = pl.ds(i * 16, 16)
            q_v[sl] = 1.0 / q_v[sl]

    xrs = (xr0, xr1, xr2, xr3)
    sems = (s0, s1, s2, s3)

    def _gather(c, slot):
        pltpu.async_copy(
            x_hbm.at[cid].at[col_v.at[c]], xrs[slot], sems[slot])

    def _wait_slot(c, slot):
        pltpu.make_async_copy(
            x_hbm.at[cid].at[col_v.at[c]], xrs[slot], sems[slot]).wait()

    def _p2chunk(blk, c, slot, prefetch, drain_first):
        if prefetch:
            if drain_first:
                _wait_slot(c, (slot + 2) % 4)
            _gather(c + 2, (slot + 2) % 4)
        _wait_slot(c, slot)
        xr = xrs[slot]
        g = blk * RB + c
        for v in range(CHUNK // 16):
            sl = pl.ds(v * 16, 16)
            rv = row_v[c, sl]
            iv = plsc.load_gather(q_v, [rv])
            w_v[sl] = w2_v[g, sl] * iv

        @plsc.parallel_loop(0, CHUNK, unroll=4)
        def _scale(e):
            cb = plsc.load_gather(w_v, [lax.broadcast(e, (16,))])
            for v in range(HD // 16):
                sl = pl.ds(v * 16, 16)
                xr[e, sl] = xr[e, sl] * cb
        pltpu.async_copy(xr, acc_sh.at[row_v.at[c]], sems[slot], add=True)

    def _p2blk(blk):
        bb = base + blk * RB
        pltpu.sync_copy(row_hbm.at[pl.ds(bb, RB)], row_v)
        pltpu.sync_copy(col_hbm.at[pl.ds(bb, RB)], col_v)
        _gather(0, 0)
        _gather(1, 1)
        for b in range(4):
            _p2chunk(blk, b, b, prefetch=True, drain_first=(b >= 2))

        def _p2(i, _2):
            for b in range(4):
                _p2chunk(blk, i * 4 + b, b, prefetch=True, drain_first=True)
            return 0

        lax.fori_loop(1, RB // 4 - 1, _p2, 0)
        for b in range(4):
            _p2chunk(blk, RB - 4 + b, b, prefetch=(b < 2), drain_first=(b < 2))
        for b in range(4):
            _wait_slot(RB - 4 + b, b)

    with jax.named_scope("sc_pass2"):
        for blk in range(NBLK):
            _p2blk(blk)
        plsc.subcore_barrier()

    wb = sid * WB_STRIDE
    nwb = WB_WIN // CHUNK

    def _wb_slice(t):
        return pl.ds(wb + t * CHUNK, CHUNK)

    def _wb_buf(t):
        return xr0 if t % 2 == 0 else xr1

    with jax.named_scope("sc_wb"):
        for t in range(nwb):
            if t >= 2:
                pltpu.make_async_copy(
                    _wb_buf(t - 2), out_hbm.at[cid].at[_wb_slice(t - 2)],
                    sem_g).wait()
            pltpu.sync_copy(acc_sh.at[_wb_slice(t)], _wb_buf(t))
            pltpu.async_copy(
                _wb_buf(t), out_hbm.at[cid].at[_wb_slice(t)], sem_g)
        for t in range(nwb - 2, nwb):
            pltpu.make_async_copy(
                _wb_buf(t), out_hbm.at[cid].at[_wb_slice(t)], sem_g).wait()


def kernel(node_features, edge_index, is_training, W_values, w_query, w_key, b):
    f32 = jnp.float32
    wqk = jnp.concatenate(
        [w_query, w_key, jnp.zeros((D, D - 2), f32)], axis=1)

    x2, qk = pl.pallas_call(
        _mm_body,
        grid=(N // MBLK,),
        in_specs=[
            pl.BlockSpec((MBLK, D), lambda i: (i, 0)),
            pl.BlockSpec((D, D), lambda i: (0, 0)),
            pl.BlockSpec((D, D), lambda i: (0, 0)),
        ],
        out_specs=[
            pl.BlockSpec((NCORE, MBLK, HD), lambda i: (0, i, 0)),
            pl.BlockSpec((MBLK, D), lambda i: (i, 0)),
        ],
        out_shape=[
            jax.ShapeDtypeStruct((NCORE, N, HD), f32),
            jax.ShapeDtypeStruct((N, D), f32),
        ],
    )(node_features, W_values, wqk)

    q = jnp.pad(qk[:, 0], (0, NPAD - N))
    k = jnp.pad(qk[:, 1], (0, NPAD - N))

    npad = EPAD - E
    row = jnp.concatenate(
        [edge_index[0], jnp.full((npad,), DUMMY, jnp.int32)]).reshape(
            NCHUNKS, CHUNK)
    col = jnp.concatenate(
        [edge_index[1], jnp.zeros((npad,), jnp.int32)]).reshape(
            NCHUNKS, CHUNK)

    parts = _sc_gat(x2, q, k, row, col)

    out = pl.pallas_call(
        _ep_body,
        grid=(N // MBLK,),
        in_specs=[
            pl.BlockSpec((NCORE, MBLK, HD), lambda i: (0, i, 0)),
            pl.BlockSpec((1, D), lambda i: (0, 0)),
        ],
        out_specs=pl.BlockSpec((MBLK, D), lambda i: (i, 0)),
        out_shape=jax.ShapeDtypeStruct((N, D), f32),
    )(parts, b.reshape(1, D))
    return out
